# baseline (speedup 1.0000x reference)
import functools

import jax
import jax.numpy as jnp
from jax import lax
from jax.experimental import pallas as pl
from jax.experimental.pallas import tpu as pltpu
from jax.experimental.pallas import tpu_sc as plsc

N_NODES = 100000
D = 128
N_SEG = 256
NC = 2
NS = 16
W = NC * NS
L = 16
TPC = N_NODES // NS
CHUNK = 6264
LB_COLS = 272
SEARCH_ITERS = 13
SEG_PER_W = N_SEG // W
BLK = 320
BADV = BLK - 8

_mesh = plsc.VectorSubcoreMesh(
    core_axis_name="c", subcore_axis_name="s", num_cores=NC, num_subcores=NS
)
_params = pltpu.CompilerParams(needs_layout_passes=False)


@functools.partial(
    pl.kernel,
    out_type=jax.ShapeDtypeStruct((N_SEG, D), jnp.float32),
    mesh=_mesh,
    compiler_params=_params,
    scratch_types=[
        pltpu.VMEM((CHUNK,), jnp.int32),
        pltpu.VMEM((LB_COLS,), jnp.int32),
        pltpu.VMEM((NS * LB_COLS,), jnp.int32),
        pltpu.VMEM((BLK, D), jnp.float32),
        pltpu.VMEM((BLK, D), jnp.float32),
        pltpu.VMEM((SEG_PER_W, D), jnp.float32),
        pltpu.VMEM((SEG_PER_W, D), jnp.float32),
        pltpu.VMEM_SHARED((NS * LB_COLS,), jnp.int32),
        pltpu.SemaphoreType.DMA,
        pltpu.SemaphoreType.DMA,
    ],
)
def _fused(
    seg_hbm, feat_hbm, out_hbm,
    ids_v, row_v, lb_v, fbuf0, fbuf1, acc_v, out_v, shared_lb, sem0, sem1,
):
    cidx = lax.axis_index("c")
    sidx = lax.axis_index("s")
    w = sidx * NC + cidx
    iota = lax.iota(jnp.int32, L)

    tlo = sidx * TPC
    cs = pl.multiple_of(jnp.minimum((tlo // 8) * 8, N_NODES - CHUNK), 8)
    pltpu.sync_copy(seg_hbm.at[pl.ds(cs, CHUNK)], ids_v)
    base = tlo - cs
    for v in range(LB_COLS // L):
        sv = v * L + iota
        lo0 = jnp.full((L,), base, jnp.int32)
        size0 = jnp.full((L,), TPC, jnp.int32)

        def step(_, carry, sv=sv):
            lo, size = carry
            active = size > 0
            half = size // 2
            mid = lo + half
            val = plsc.load_gather(ids_v, [jnp.minimum(mid, CHUNK - 1)])
            pred = active & (val < sv)
            lo = jnp.where(pred, mid + 1, lo)
            size = jnp.where(active, jnp.where(pred, size - half - 1, half), size)
            return lo, size

        lo, _unused = lax.fori_loop(0, SEARCH_ITERS, step, (lo0, size0))
        row_v[pl.ds(v * L, L)] = lo - base
    pltpu.sync_copy(
        row_v, shared_lb.at[pl.ds(pl.multiple_of(sidx * LB_COLS, 8), LB_COLS)]
    )
    plsc.subcore_barrier()
    pltpu.sync_copy(shared_lb, lb_v)

    col = SEG_PER_W * w + iota

    def sum_row(r, acc):
        return acc + plsc.load_gather(lb_v, [r * LB_COLS + col])

    bsum = lax.fori_loop(0, NS, sum_row, jnp.zeros((L,), jnp.int32))
    b = [jnp.sum(jnp.where(iota == j, bsum, 0)) for j in range(SEG_PER_W + 1)]

    zf = jnp.zeros((L,), jnp.float32)
    for j in range(SEG_PER_W):
        for c in range(D // L):
            acc_v[j, pl.ds(c * L, L)] = zf

    b_lo, b_hi = b[0], b[SEG_PER_W]
    nblk = (b_hi - b_lo + BADV - 1) // BADV

    def blk_start(k):
        blk_lo = b_lo + k * BADV
        return pl.multiple_of(jnp.minimum((blk_lo // 8) * 8, N_NODES - BLK), 8)

    def dma_desc(k, fb, sem):
        return pltpu.make_async_copy(
            feat_hbm.at[pl.ds(blk_start(k), BLK)], fb, sem
        )

    def process(k, fb):
        blk_lo = b_lo + k * BADV
        blk_hi = jnp.minimum(blk_lo + BADV, b_hi)
        start = blk_start(k)
        for j in range(SEG_PER_W):
            lo = jnp.maximum(b[j], blk_lo)
            hi = jnp.minimum(b[j + 1], blk_hi)

            @pl.when(hi > lo)
            def _(j=j, lo=lo, hi=hi, start=start, fb=fb):
                ilo = lo - start
                ihi = hi - start
                n = ihi - ilo

                def pair_body(p, accs):
                    i = ilo + 2 * p
                    return tuple(
                        accs[c] + (fb[i, pl.ds(c * L, L)] + fb[i + 1, pl.ds(c * L, L)])
                        for c in range(D // L)
                    )

                accs = lax.fori_loop(
                    0, n // 2, pair_body, tuple(zf for _ in range(D // L))
                )
                tailm = jnp.full((L,), n % 2, jnp.int32).astype(jnp.float32)
                accs = tuple(
                    accs[c] + fb[ihi - 1, pl.ds(c * L, L)] * tailm
                    for c in range(D // L)
                )
                for c in range(D // L):
                    plsc.addupdate(acc_v.at[j, pl.ds(c * L, L)], accs[c])

    @pl.when(nblk > 0)
    def _():
        dma_desc(0, fbuf0, sem0).start()

    def pair_body(p, carry):
        k0 = 2 * p

        @pl.when(k0 < nblk)
        def _():
            dma_desc(k0, fbuf0, sem0).wait()

            @pl.when(k0 + 1 < nblk)
            def _():
                dma_desc(k0 + 1, fbuf1, sem1).start()

            process(k0, fbuf0)

        @pl.when(k0 + 1 < nblk)
        def _():
            dma_desc(k0 + 1, fbuf1, sem1).wait()

            @pl.when(k0 + 2 < nblk)
            def _():
                dma_desc(k0 + 2, fbuf0, sem0).start()

            process(k0 + 1, fbuf1)

        return carry

    lax.fori_loop(0, (nblk + 1) // 2, pair_body, 0)

    for j in range(SEG_PER_W):
        cntv = jnp.full((L,), b[j + 1] - b[j], jnp.int32).astype(jnp.float32)
        rec = 1.0 / jnp.maximum(cntv, 1.0)
        for c in range(D // L):
            out_v[j, pl.ds(c * L, L)] = acc_v[j, pl.ds(c * L, L)] * rec
    pltpu.sync_copy(
        out_v, out_hbm.at[pl.ds(pl.multiple_of(SEG_PER_W * w, 8), SEG_PER_W)]
    )


def kernel(feat, segment_ids):
    seg = segment_ids.astype(jnp.int32)
    return _fused(seg, feat)


# speedup vs baseline: 1.0112x; 1.0112x over previous
import functools

import jax
import jax.numpy as jnp
from jax import lax
from jax.experimental import pallas as pl
from jax.experimental.pallas import tpu as pltpu
from jax.experimental.pallas import tpu_sc as plsc

N_NODES = 100000
D = 128
N_SEG = 256
NC = 2
NS = 16
W = NC * NS
L = 16
TPC = N_NODES // NS
CHUNK = 6264
LB_COLS = 272
SEARCH_ITERS = 13
SEG_PER_W = N_SEG // W
BLK = 320
BADV = BLK - 8

R0 = 50000
TC_BLK = 2000
TC_STEPS = (N_NODES - R0) // TC_BLK

_mesh = plsc.VectorSubcoreMesh(
    core_axis_name="c", subcore_axis_name="s", num_cores=NC, num_subcores=NS
)
_params = pltpu.CompilerParams(needs_layout_passes=False)


@functools.partial(
    pl.kernel,
    out_type=(
        jax.ShapeDtypeStruct((N_SEG, D), jnp.float32),
        jax.ShapeDtypeStruct((N_SEG, D), jnp.float32),
    ),
    mesh=_mesh,
    compiler_params=_params,
    scratch_types=[
        pltpu.VMEM((CHUNK,), jnp.int32),
        pltpu.VMEM((LB_COLS,), jnp.int32),
        pltpu.VMEM((NS * LB_COLS,), jnp.int32),
        pltpu.VMEM((BLK, D), jnp.float32),
        pltpu.VMEM((BLK, D), jnp.float32),
        pltpu.VMEM((SEG_PER_W, D), jnp.float32),
        pltpu.VMEM((SEG_PER_W, D), jnp.float32),
        pltpu.VMEM_SHARED((NS * LB_COLS,), jnp.int32),
        pltpu.SemaphoreType.DMA,
        pltpu.SemaphoreType.DMA,
    ],
)
def _fused(
    seg_hbm, feat_hbm, sums_hbm, rec_hbm,
    ids_v, row_v, lb_v, fbuf0, fbuf1, acc_v, out_v, shared_lb, sem0, sem1,
):
    cidx = lax.axis_index("c")
    sidx = lax.axis_index("s")
    w = sidx * NC + cidx
    iota = lax.iota(jnp.int32, L)

    tlo = sidx * TPC
    cs = pl.multiple_of(jnp.minimum((tlo // 8) * 8, N_NODES - CHUNK), 8)
    pltpu.sync_copy(seg_hbm.at[pl.ds(cs, CHUNK)], ids_v)
    base = tlo - cs
    for v in range(LB_COLS // L):
        sv = v * L + iota
        lo0 = jnp.full((L,), base, jnp.int32)
        size0 = jnp.full((L,), TPC, jnp.int32)

        def step(_, carry, sv=sv):
            lo, size = carry
            active = size > 0
            half = size // 2
            mid = lo + half
            val = plsc.load_gather(ids_v, [jnp.minimum(mid, CHUNK - 1)])
            pred = active & (val < sv)
            lo = jnp.where(pred, mid + 1, lo)
            size = jnp.where(active, jnp.where(pred, size - half - 1, half), size)
            return lo, size

        lo, _unused = lax.fori_loop(0, SEARCH_ITERS, step, (lo0, size0))
        row_v[pl.ds(v * L, L)] = lo - base
    pltpu.sync_copy(
        row_v, shared_lb.at[pl.ds(pl.multiple_of(sidx * LB_COLS, 8), LB_COLS)]
    )
    plsc.subcore_barrier()
    pltpu.sync_copy(shared_lb, lb_v)

    col = SEG_PER_W * w + iota

    def sum_row(r, acc):
        return acc + plsc.load_gather(lb_v, [r * LB_COLS + col])

    bsum = lax.fori_loop(0, NS, sum_row, jnp.zeros((L,), jnp.int32))
    b = [jnp.sum(jnp.where(iota == j, bsum, 0)) for j in range(SEG_PER_W + 1)]
    bc = [jnp.minimum(bj, R0) for bj in b]

    zf = jnp.zeros((L,), jnp.float32)
    for j in range(SEG_PER_W):
        for c in range(D // L):
            acc_v[j, pl.ds(c * L, L)] = zf

    b_lo, b_hi = bc[0], bc[SEG_PER_W]
    nblk = (b_hi - b_lo + BADV - 1) // BADV

    def blk_start(k):
        blk_lo = b_lo + k * BADV
        return pl.multiple_of(jnp.minimum((blk_lo // 8) * 8, N_NODES - BLK), 8)

    def dma_desc(k, fb, sem):
        return pltpu.make_async_copy(
            feat_hbm.at[pl.ds(blk_start(k), BLK)], fb, sem
        )

    def process(k, fb):
        blk_lo = b_lo + k * BADV
        blk_hi = jnp.minimum(blk_lo + BADV, b_hi)
        start = blk_start(k)
        for j in range(SEG_PER_W):
            lo = jnp.maximum(bc[j], blk_lo)
            hi = jnp.minimum(bc[j + 1], blk_hi)

            @pl.when(hi > lo)
            def _(j=j, lo=lo, hi=hi, start=start, fb=fb):
                ilo = lo - start
                ihi = hi - start

                def row_body(i, accs):
                    return tuple(
                        accs[c] + fb[i, pl.ds(c * L, L)] for c in range(D // L)
                    )

                accs = lax.fori_loop(
                    ilo, ihi, row_body, tuple(zf for _ in range(D // L))
                )
                for c in range(D // L):
                    plsc.addupdate(acc_v.at[j, pl.ds(c * L, L)], accs[c])

    @pl.when(nblk > 0)
    def _():
        dma_desc(0, fbuf0, sem0).start()

    def pair_body(p, carry):
        k0 = 2 * p

        @pl.when(k0 < nblk)
        def _():
            dma_desc(k0, fbuf0, sem0).wait()

            @pl.when(k0 + 1 < nblk)
            def _():
                dma_desc(k0 + 1, fbuf1, sem1).start()

            process(k0, fbuf0)

        @pl.when(k0 + 1 < nblk)
        def _():
            dma_desc(k0 + 1, fbuf1, sem1).wait()

            @pl.when(k0 + 2 < nblk)
            def _():
                dma_desc(k0 + 2, fbuf0, sem0).start()

            process(k0 + 1, fbuf1)

        return carry

    lax.fori_loop(0, (nblk + 1) // 2, pair_body, 0)

    pltpu.sync_copy(
        acc_v, sums_hbm.at[pl.ds(pl.multiple_of(SEG_PER_W * w, 8), SEG_PER_W)]
    )
    for j in range(SEG_PER_W):
        cntv = jnp.full((L,), b[j + 1] - b[j], jnp.int32).astype(jnp.float32)
        rec = 1.0 / jnp.maximum(cntv, 1.0)
        for c in range(D // L):
            out_v[j, pl.ds(c * L, L)] = rec
    pltpu.sync_copy(
        out_v, rec_hbm.at[pl.ds(pl.multiple_of(SEG_PER_W * w, 8), SEG_PER_W)]
    )


def _tc_tail_body(seg_ref, feat_ref, out_ref):
    k = pl.program_id(0)
    ids = seg_ref[0, 0, :].reshape(TC_BLK, 1)
    onehot = (
        ids == lax.broadcasted_iota(jnp.int32, (TC_BLK, N_SEG), 1)
    ).astype(jnp.float32)
    part = lax.dot_general(
        onehot,
        feat_ref[...],
        dimension_numbers=(((0,), (0,)), ((), ())),
        preferred_element_type=jnp.float32,
    )

    @pl.when(k == 0)
    def _():
        out_ref[...] = part

    @pl.when(k > 0)
    def _():
        out_ref[...] += part


_tc_tail = pl.pallas_call(
    _tc_tail_body,
    grid=(TC_STEPS,),
    in_specs=[
        pl.BlockSpec((1, 1, TC_BLK), lambda k: (R0 // TC_BLK + k, 0, 0)),
        pl.BlockSpec((TC_BLK, D), lambda k: (R0 // TC_BLK + k, 0)),
    ],
    out_specs=pl.BlockSpec((N_SEG, D), lambda k: (0, 0)),
    out_shape=jax.ShapeDtypeStruct((N_SEG, D), jnp.float32),
)


def _combine_body(a_ref, b_ref, r_ref, out_ref):
    out_ref[...] = (a_ref[...] + b_ref[...]) * r_ref[...]


_combine = pl.pallas_call(
    _combine_body,
    out_shape=jax.ShapeDtypeStruct((N_SEG, D), jnp.float32),
)


def kernel(feat, segment_ids):
    seg = segment_ids.astype(jnp.int32)
    sc_sums, rec = _fused(seg, feat)
    seg3 = seg.reshape(N_NODES // TC_BLK, 1, TC_BLK)
    tc_sums = _tc_tail(seg3, feat)
    return _combine(sc_sums, tc_sums, rec)


# speedup vs baseline: 1.0578x; 1.0461x over previous
import functools

import jax
import jax.numpy as jnp
from jax import lax
from jax.experimental import pallas as pl
from jax.experimental.pallas import tpu as pltpu
from jax.experimental.pallas import tpu_sc as plsc

N_NODES = 100000
D = 128
N_SEG = 256
NC = 2
NS = 16
W = NC * NS
L = 16
TPC = N_NODES // NS
CHUNK = 6264
LB_COLS = 272
SEARCH_ITERS = 13
SEG_PER_W = N_SEG // W
BLK = 320
BADV = BLK - 8

_mesh = plsc.VectorSubcoreMesh(
    core_axis_name="c", subcore_axis_name="s", num_cores=NC, num_subcores=NS
)
_params = pltpu.CompilerParams(needs_layout_passes=False)


@functools.partial(
    pl.kernel,
    out_type=jax.ShapeDtypeStruct((N_SEG, D), jnp.float32),
    mesh=_mesh,
    compiler_params=_params,
    scratch_types=[
        pltpu.VMEM((CHUNK,), jnp.int32),
        pltpu.VMEM((LB_COLS,), jnp.int32),
        pltpu.VMEM((NS * LB_COLS,), jnp.int32),
        pltpu.VMEM((BLK, D), jnp.float32),
        pltpu.VMEM((BLK, D), jnp.float32),
        pltpu.VMEM((SEG_PER_W, D), jnp.float32),
        pltpu.VMEM((SEG_PER_W, D), jnp.float32),
        pltpu.VMEM_SHARED((NS * LB_COLS,), jnp.int32),
        pltpu.SemaphoreType.DMA,
        pltpu.SemaphoreType.DMA,
    ],
)
def _fused(
    seg_hbm, feat_hbm, out_hbm,
    ids_v, row_v, lb_v, fbuf0, fbuf1, acc_v, out_v, shared_lb, sem0, sem1,
):
    cidx = lax.axis_index("c")
    sidx = lax.axis_index("s")
    w = sidx * NC + cidx
    iota = lax.iota(jnp.int32, L)

    tlo = sidx * TPC
    cs = pl.multiple_of(jnp.minimum((tlo // 8) * 8, N_NODES - CHUNK), 8)
    pltpu.sync_copy(seg_hbm.at[pl.ds(cs, CHUNK)], ids_v)
    base = tlo - cs
    for v in range(LB_COLS // L):
        sv = v * L + iota
        lo0 = jnp.full((L,), base, jnp.int32)
        size0 = jnp.full((L,), TPC, jnp.int32)

        def step(_, carry, sv=sv):
            lo, size = carry
            active = size > 0
            half = size // 2
            mid = lo + half
            val = plsc.load_gather(ids_v, [jnp.minimum(mid, CHUNK - 1)])
            pred = active & (val < sv)
            lo = jnp.where(pred, mid + 1, lo)
            size = jnp.where(active, jnp.where(pred, size - half - 1, half), size)
            return lo, size

        lo, _unused = lax.fori_loop(0, SEARCH_ITERS, step, (lo0, size0))
        row_v[pl.ds(v * L, L)] = lo - base
    pltpu.sync_copy(
        row_v, shared_lb.at[pl.ds(pl.multiple_of(sidx * LB_COLS, 8), LB_COLS)]
    )
    plsc.subcore_barrier()
    pltpu.sync_copy(shared_lb, lb_v)

    col = SEG_PER_W * w + iota

    def sum_row(r, acc):
        return acc + plsc.load_gather(lb_v, [r * LB_COLS + col])

    bsum = lax.fori_loop(0, NS, sum_row, jnp.zeros((L,), jnp.int32))
    b = [jnp.sum(jnp.where(iota == j, bsum, 0)) for j in range(SEG_PER_W + 1)]

    zf = jnp.zeros((L,), jnp.float32)
    for j in range(SEG_PER_W):
        for c in range(D // L):
            acc_v[j, pl.ds(c * L, L)] = zf

    b_lo, b_hi = b[0], b[SEG_PER_W]
    nblk = (b_hi - b_lo + BADV - 1) // BADV

    def blk_start(k):
        blk_lo = b_lo + k * BADV
        return pl.multiple_of(jnp.minimum((blk_lo // 8) * 8, N_NODES - BLK), 8)

    def dma_desc(k, fb, sem):
        return pltpu.make_async_copy(
            feat_hbm.at[pl.ds(blk_start(k), BLK)], fb, sem
        )

    def process(k, fb):
        blk_lo = b_lo + k * BADV
        blk_hi = jnp.minimum(blk_lo + BADV, b_hi)
        start = blk_start(k)
        for j in range(SEG_PER_W):
            lo = jnp.maximum(b[j], blk_lo)
            hi = jnp.minimum(b[j + 1], blk_hi)

            @pl.when(hi > lo)
            def _(j=j, lo=lo, hi=hi, start=start, fb=fb):
                ilo = lo - start
                ihi = hi - start

                def row_body(i, accs):
                    return tuple(
                        accs[c] + fb[i, pl.ds(c * L, L)] for c in range(D // L)
                    )

                accs = lax.fori_loop(
                    ilo, ihi, row_body, tuple(zf for _ in range(D // L))
                )
                for c in range(D // L):
                    plsc.addupdate(acc_v.at[j, pl.ds(c * L, L)], accs[c])

    @pl.when(nblk > 0)
    def _():
        dma_desc(0, fbuf0, sem0).start()

    def pair_body(p, carry):
        k0 = 2 * p

        @pl.when(k0 < nblk)
        def _():
            dma_desc(k0, fbuf0, sem0).wait()

            @pl.when(k0 + 1 < nblk)
            def _():
                dma_desc(k0 + 1, fbuf1, sem1).start()

            process(k0, fbuf0)

        @pl.when(k0 + 1 < nblk)
        def _():
            dma_desc(k0 + 1, fbuf1, sem1).wait()

            @pl.when(k0 + 2 < nblk)
            def _():
                dma_desc(k0 + 2, fbuf0, sem0).start()

            process(k0 + 1, fbuf1)

        return carry

    lax.fori_loop(0, (nblk + 1) // 2, pair_body, 0)

    for j in range(SEG_PER_W):
        cntv = jnp.full((L,), b[j + 1] - b[j], jnp.int32).astype(jnp.float32)
        rec = 1.0 / jnp.maximum(cntv, 1.0)
        for c in range(D // L):
            out_v[j, pl.ds(c * L, L)] = acc_v[j, pl.ds(c * L, L)] * rec
    pltpu.sync_copy(
        out_v, out_hbm.at[pl.ds(pl.multiple_of(SEG_PER_W * w, 8), SEG_PER_W)]
    )


def kernel(feat, segment_ids):
    seg = segment_ids.astype(jnp.int32)
    return _fused(seg, feat)
